# baseline (speedup 1.0000x reference)
import dataclasses
import functools

import jax
import jax.numpy as jnp
from jax import lax
from jax.experimental import pallas as pl
from jax.experimental.pallas import tpu as pltpu
from jax.experimental.pallas import tpu_sc as plsc

NC = 2
NS = 16
NW = NC * NS
LANES = 16
K = 128
W = 8

_MESH = plsc.VectorSubcoreMesh(core_axis_name="c", subcore_axis_name="s")

_CP = pltpu.CompilerParams()
if "needs_layout_passes" in pltpu.CompilerParams.__dataclass_fields__:
    _CP = dataclasses.replace(_CP, needs_layout_passes=False)


def _sc_deg(dst_t, ew_t, n_pad):
    nb = dst_t.shape[1]

    @functools.partial(
        pl.kernel,
        out_type=jax.ShapeDtypeStruct((NW, n_pad), jnp.float32),
        mesh=_MESH,
        compiler_params=_CP,
        scratch_types=[
            pltpu.VMEM((nb, K), jnp.int32),
            pltpu.VMEM((nb, K), jnp.float32),
            pltpu.VMEM((n_pad,), jnp.float32),
        ],
    )
    def k(dst_hbm, ew_hbm, out_hbm, dst_v, ew_v, deg_v):
        c = lax.axis_index("c")
        s = lax.axis_index("s")
        wid = c * NS + s
        pltpu.sync_copy(dst_hbm.at[wid], dst_v)
        pltpu.sync_copy(ew_hbm.at[wid], ew_v)

        @pl.loop(0, n_pad, step=LANES)
        def _(i):
            deg_v[pl.ds(i, LANES)] = jnp.zeros((LANES,), jnp.float32)

        @pl.loop(0, nb)
        def _(j):
            @pl.loop(0, K, step=LANES)
            def _(i):
                idx = dst_v[j, pl.ds(i, LANES)]
                val = ew_v[j, pl.ds(i, LANES)]
                plsc.addupdate_scatter(deg_v, [idx], val)

        pltpu.sync_copy(deg_v, out_hbm.at[wid])

    return k(dst_t, ew_t)


def _sc_norm(src_t, dst_t, ew_t, dis, n_pad):
    nb = src_t.shape[1]

    @functools.partial(
        pl.kernel,
        out_type=jax.ShapeDtypeStruct(src_t.shape, jnp.float32),
        mesh=_MESH,
        compiler_params=_CP,
        scratch_types=[
            pltpu.VMEM((nb, K), jnp.int32),
            pltpu.VMEM((nb, K), jnp.int32),
            pltpu.VMEM((nb, K), jnp.float32),
            pltpu.VMEM((n_pad,), jnp.float32),
        ],
    )
    def k(src_hbm, dst_hbm, ew_hbm, dis_hbm, out_hbm, src_v, dst_v, ew_v, dis_v):
        c = lax.axis_index("c")
        s = lax.axis_index("s")
        wid = c * NS + s
        pltpu.sync_copy(src_hbm.at[wid], src_v)
        pltpu.sync_copy(dst_hbm.at[wid], dst_v)
        pltpu.sync_copy(ew_hbm.at[wid], ew_v)
        pltpu.sync_copy(dis_hbm, dis_v)

        @pl.loop(0, nb)
        def _(j):
            @pl.loop(0, K, step=LANES)
            def _(i):
                si = src_v[j, pl.ds(i, LANES)]
                di = dst_v[j, pl.ds(i, LANES)]
                a = plsc.load_gather(dis_v, [si])
                b = plsc.load_gather(dis_v, [di])
                ew_v[j, pl.ds(i, LANES)] = a * b * ew_v[j, pl.ds(i, LANES)]

        pltpu.sync_copy(ew_v, out_hbm.at[wid])

    return k(src_t, dst_t, ew_t, dis)


def _sc_agg(h, src_t, dst_t, norm_t, n_pad):
    nb = src_t.shape[1]
    d = h.shape[1]
    rows_per_tile = n_pad // NS
    nw = nb // W

    @functools.partial(
        pl.kernel,
        out_type=jax.ShapeDtypeStruct((NC, n_pad, d), jnp.float32),
        mesh=_MESH,
        compiler_params=_CP,
        scratch_types=[
            pltpu.VMEM((W, K), jnp.int32),
            pltpu.VMEM((W, K), jnp.int32),
            pltpu.VMEM((W, K), jnp.int32),
            pltpu.VMEM((W, K), jnp.int32),
            pltpu.VMEM((W, K), jnp.float32),
            pltpu.VMEM((W, K), jnp.float32),
            pltpu.VMEM((K, d), jnp.float32),
            pltpu.VMEM((K, d), jnp.float32),
            pltpu.VMEM_SHARED((n_pad, d), jnp.float32),
            pltpu.SemaphoreType.DMA,
            pltpu.SemaphoreType.DMA,
            pltpu.SemaphoreType.DMA,
            pltpu.SemaphoreType.DMA,
            pltpu.SemaphoreType.DMA,
            pltpu.SemaphoreType.DMA,
        ],
    )
    def k(h_hbm, src_hbm, dst_hbm, norm_hbm, out_hbm,
          src_a, src_b, dst_a, dst_b, nrm_a, nrm_b, rows_a, rows_b, acc_sh,
          sga, sgb, ssa, ssb, sia, sib):
        c = lax.axis_index("c")
        s = lax.axis_index("s")
        wid = c * NS + s
        bufs = ((src_a, dst_a, nrm_a, sia), (src_b, dst_b, nrm_b, sib))
        rows = ((rows_a, sga, ssa), (rows_b, sgb, ssb))

        def idx_copies(w, parity):
            sw, dw_, nw_, si = bufs[parity]
            return (
                pltpu.make_async_copy(src_hbm.at[wid, pl.ds(w * W, W)], sw, si),
                pltpu.make_async_copy(dst_hbm.at[wid, pl.ds(w * W, W)], dw_, si),
                pltpu.make_async_copy(norm_hbm.at[wid, pl.ds(w * W, W)], nw_, si),
            )

        def fetch_idx(w, parity):
            for cp in idx_copies(w, parity):
                cp.start()

        def wait_idx(w, parity):
            for cp in idx_copies(w, parity):
                cp.wait()

        @pl.loop(0, K)
        def _(r):
            for q in range(d // LANES):
                rows_b[r, pl.ds(q * LANES, LANES)] = jnp.zeros((LANES,), jnp.float32)

        @pl.loop(0, rows_per_tile, step=K)
        def _(r):
            pltpu.sync_copy(rows_b, acc_sh.at[pl.ds(s * rows_per_tile + r, K), :])

        for cp in idx_copies(0, 0):
            cp.start()
        wait_idx(0, 0)
        fetch_idx(1, 1)
        pltpu.async_copy(h_hbm.at[src_a.at[0]], rows_a, sga)

        plsc.subcore_barrier()

        def scale(buf, nrm, b):
            @pl.loop(0, K, step=LANES)
            def _(i):
                nv = nrm[b, pl.ds(i, LANES)]
                for t in range(LANES):
                    sc = nv[t]
                    for q in range(d // LANES):
                        sl = pl.ds(q * LANES, LANES)
                        buf[i + t, sl] = buf[i + t, sl] * sc

        @pl.loop(0, nw, step=2)
        def _(w2):
            for dw in (0, 1):
                sw, dw_, nw_, _si = bufs[dw]
                for b in range(W):
                    cur, sg_c, ss_c = rows[b % 2]
                    nxt, sg_n, ss_n = rows[1 - b % 2]
                    pltpu.make_async_copy(h_hbm.at[sw.at[b]], cur, sg_c).wait()
                    if b < W - 1:
                        if b >= 1:
                            pltpu.make_async_copy(
                                nxt, acc_sh.at[dw_.at[b - 1]], ss_n).wait()
                        pltpu.async_copy(h_hbm.at[sw.at[b + 1]], nxt, sg_n)
                    scale(cur, nw_, b)
                    pltpu.async_copy(cur, acc_sh.at[dw_.at[b]], ss_c, add=True)

                pltpu.make_async_copy(rows_a, acc_sh.at[dw_.at[W - 2]], ssa).wait()
                pltpu.make_async_copy(rows_b, acc_sh.at[dw_.at[W - 1]], ssb).wait()
                if dw == 0:
                    wait_idx(w2 + 1, 1)
                    pltpu.async_copy(h_hbm.at[src_b.at[0]], rows_a, sga)

                    @pl.when(w2 + 2 < nw)
                    def _():
                        fetch_idx(w2 + 2, 0)
                else:
                    @pl.when(w2 + 2 < nw)
                    def _():
                        wait_idx(w2 + 2, 0)
                        pltpu.async_copy(h_hbm.at[src_a.at[0]], rows_a, sga)

                    @pl.when(w2 + 3 < nw)
                    def _():
                        fetch_idx(w2 + 3, 1)

        plsc.subcore_barrier()
        pltpu.sync_copy(
            acc_sh.at[pl.ds(s * rows_per_tile, rows_per_tile), :],
            out_hbm.at[c, pl.ds(s * rows_per_tile, rows_per_tile), :],
        )

    return k(h, src_t, dst_t, norm_t)


def _tc_finalize_deg(deg_parts_t):
    n_pad = deg_parts_t.shape[0]

    def body(p_ref, dis_ref):
        deg = 1.0 + jnp.sum(p_ref[...], axis=1, keepdims=True)
        dis_ref[...] = lax.rsqrt(deg)

    return pl.pallas_call(
        body,
        out_shape=jax.ShapeDtypeStruct((n_pad, 1), jnp.float32),
    )(deg_parts_t)


def _tc_matmul(x, w):
    n, d_in = x.shape
    d_out = w.shape[1]
    bn = 1280

    def body(x_ref, w_ref, o_ref):
        o_ref[...] = jnp.dot(x_ref[...], w_ref[...],
                             preferred_element_type=jnp.float32)

    return pl.pallas_call(
        body,
        grid=(n // bn,),
        in_specs=[
            pl.BlockSpec((bn, d_in), lambda i: (i, 0)),
            pl.BlockSpec((d_in, d_out), lambda i: (0, 0)),
        ],
        out_specs=pl.BlockSpec((bn, d_out), lambda i: (i, 0)),
        out_shape=jax.ShapeDtypeStruct((n, d_out), jnp.float32),
    )(x, w)


def _tc_combine_mm(p0, p1, h, dis2d, b2d, w):
    n, d = h.shape
    d_out = w.shape[1]
    bn = 1280

    def body(p0_ref, p1_ref, h_ref, dis_ref, b_ref, w_ref, o_ref):
        inv_deg = dis_ref[...] * dis_ref[...]
        act = p0_ref[...] + p1_ref[...] + h_ref[...] * inv_deg + b_ref[...]
        act = jnp.maximum(act, 0.0)
        o_ref[...] = jnp.dot(act, w_ref[...], preferred_element_type=jnp.float32)

    return pl.pallas_call(
        body,
        grid=(n // bn,),
        in_specs=[
            pl.BlockSpec((bn, d), lambda i: (i, 0)),
            pl.BlockSpec((bn, d), lambda i: (i, 0)),
            pl.BlockSpec((bn, d), lambda i: (i, 0)),
            pl.BlockSpec((bn, 1), lambda i: (i, 0)),
            pl.BlockSpec((1, d), lambda i: (0, 0)),
            pl.BlockSpec((d, d_out), lambda i: (0, 0)),
        ],
        out_specs=pl.BlockSpec((bn, d_out), lambda i: (i, 0)),
        out_shape=jax.ShapeDtypeStruct((n, d_out), jnp.float32),
    )(p0, p1, h, dis2d, b2d, w)


def _tc_final(p0, p1, h, dis2d, b2d):
    n, d = h.shape
    bn = 1280

    def body(p0_ref, p1_ref, h_ref, dis_ref, b_ref, o_ref):
        inv_deg = dis_ref[...] * dis_ref[...]
        o_ref[...] = p0_ref[...] + p1_ref[...] + h_ref[...] * inv_deg + b_ref[...]

    return pl.pallas_call(
        body,
        grid=(n // bn,),
        in_specs=[
            pl.BlockSpec((bn, d), lambda i: (i, 0)),
            pl.BlockSpec((bn, d), lambda i: (i, 0)),
            pl.BlockSpec((bn, d), lambda i: (i, 0)),
            pl.BlockSpec((bn, 1), lambda i: (i, 0)),
            pl.BlockSpec((1, d), lambda i: (0, 0)),
        ],
        out_specs=pl.BlockSpec((bn, d), lambda i: (i, 0)),
        out_shape=jax.ShapeDtypeStruct((n, d), jnp.float32),
    )(p0, p1, h, dis2d, b2d)


def kernel(x, edge_index, edge_weight, W1, b1, W2, b2, W3, b3):
    n, d = x.shape
    e = edge_weight.shape[0]

    n_pad = ((n + NS * K - 1) // (NS * K)) * (NS * K)
    eq = 2 * W * NW * K
    e_pad = ((e + eq - 1) // eq) * eq
    nb = e_pad // (NW * K)

    src, dst, ew = jax.lax.sort(
        (edge_index[0], edge_index[1], edge_weight), num_keys=1)
    pad_e = e_pad - e
    src_t = jnp.concatenate([src, jnp.zeros((pad_e,), jnp.int32)]).reshape(NW, nb, K)
    dst_t = jnp.concatenate([dst, jnp.zeros((pad_e,), jnp.int32)]).reshape(NW, nb, K)
    ew_t = jnp.concatenate(
        [ew, jnp.zeros((pad_e,), jnp.float32)]).reshape(NW, nb, K)
    x_p = jnp.pad(x, ((0, n_pad - n), (0, 0)))

    deg_parts = _sc_deg(dst_t, ew_t, n_pad)
    dis2d = _tc_finalize_deg(deg_parts.T)
    dis = dis2d.reshape(n_pad)
    norm_t = _sc_norm(src_t, dst_t, ew_t, dis, n_pad)

    b1r = b1.reshape(1, -1)
    b2r = b2.reshape(1, -1)
    b3r = b3.reshape(1, -1)

    h1 = _tc_matmul(x_p, W1)
    p = _sc_agg(h1, src_t, dst_t, norm_t, n_pad)
    h2 = _tc_combine_mm(p[0], p[1], h1, dis2d, b1r, W2)
    p = _sc_agg(h2, src_t, dst_t, norm_t, n_pad)
    h3 = _tc_combine_mm(p[0], p[1], h2, dis2d, b2r, W3)
    p = _sc_agg(h3, src_t, dst_t, norm_t, n_pad)
    out = _tc_final(p[0], p[1], h3, dis2d, b3r)
    return out[:n]


# speedup vs baseline: 1.2515x; 1.2515x over previous
import dataclasses
import functools

import jax
import jax.numpy as jnp
from jax import lax
from jax.experimental import pallas as pl
from jax.experimental.pallas import tpu as pltpu
from jax.experimental.pallas import tpu_sc as plsc

NC = 2
NS = 16
NW = NC * NS
LANES = 16
K = 128
W = 8

_MESH = plsc.VectorSubcoreMesh(core_axis_name="c", subcore_axis_name="s")

_CP = pltpu.CompilerParams()
if "needs_layout_passes" in pltpu.CompilerParams.__dataclass_fields__:
    _CP = dataclasses.replace(_CP, needs_layout_passes=False)


def _sc_deg(dst_t, ew_t, n_pad):
    nb = dst_t.shape[1]

    @functools.partial(
        pl.kernel,
        out_type=jax.ShapeDtypeStruct((NW, n_pad), jnp.float32),
        mesh=_MESH,
        compiler_params=_CP,
        scratch_types=[
            pltpu.VMEM((nb, K), jnp.int32),
            pltpu.VMEM((nb, K), jnp.float32),
            pltpu.VMEM((n_pad,), jnp.float32),
        ],
    )
    def k(dst_hbm, ew_hbm, out_hbm, dst_v, ew_v, deg_v):
        c = lax.axis_index("c")
        s = lax.axis_index("s")
        wid = c * NS + s
        pltpu.sync_copy(dst_hbm.at[wid], dst_v)
        pltpu.sync_copy(ew_hbm.at[wid], ew_v)

        @pl.loop(0, n_pad, step=LANES)
        def _(i):
            deg_v[pl.ds(i, LANES)] = jnp.zeros((LANES,), jnp.float32)

        @pl.loop(0, nb)
        def _(j):
            @pl.loop(0, K, step=LANES)
            def _(i):
                idx = dst_v[j, pl.ds(i, LANES)]
                val = ew_v[j, pl.ds(i, LANES)]
                plsc.addupdate_scatter(deg_v, [idx], val)

        pltpu.sync_copy(deg_v, out_hbm.at[wid])

    return k(dst_t, ew_t)


def _sc_norm(src_t, dst_t, ew_t, dis, n_pad):
    nb = src_t.shape[1]

    @functools.partial(
        pl.kernel,
        out_type=jax.ShapeDtypeStruct(src_t.shape, jnp.float32),
        mesh=_MESH,
        compiler_params=_CP,
        scratch_types=[
            pltpu.VMEM((nb, K), jnp.int32),
            pltpu.VMEM((nb, K), jnp.int32),
            pltpu.VMEM((nb, K), jnp.float32),
            pltpu.VMEM((n_pad,), jnp.float32),
        ],
    )
    def k(src_hbm, dst_hbm, ew_hbm, dis_hbm, out_hbm, src_v, dst_v, ew_v, dis_v):
        c = lax.axis_index("c")
        s = lax.axis_index("s")
        wid = c * NS + s
        pltpu.sync_copy(src_hbm.at[wid], src_v)
        pltpu.sync_copy(dst_hbm.at[wid], dst_v)
        pltpu.sync_copy(ew_hbm.at[wid], ew_v)
        pltpu.sync_copy(dis_hbm, dis_v)

        @pl.loop(0, nb)
        def _(j):
            @pl.loop(0, K, step=LANES)
            def _(i):
                si = src_v[j, pl.ds(i, LANES)]
                di = dst_v[j, pl.ds(i, LANES)]
                a = plsc.load_gather(dis_v, [si])
                b = plsc.load_gather(dis_v, [di])
                ew_v[j, pl.ds(i, LANES)] = a * b * ew_v[j, pl.ds(i, LANES)]

        pltpu.sync_copy(ew_v, out_hbm.at[wid])

    return k(src_t, dst_t, ew_t, dis)


def _sc_agg(h, src_t, dst_t, norm_t, n_pad):
    nb = src_t.shape[1]
    d = h.shape[1]
    rows_per_tile = n_pad // NS

    @functools.partial(
        pl.kernel,
        out_type=jax.ShapeDtypeStruct((NC, n_pad, d), jnp.float32),
        mesh=_MESH,
        compiler_params=_CP,
        scratch_types=[
            pltpu.VMEM((nb, K), jnp.int32),
            pltpu.VMEM((nb, K), jnp.int32),
            pltpu.VMEM((nb, K), jnp.float32),
            pltpu.VMEM((K, d), jnp.float32),
            pltpu.VMEM_SHARED((n_pad, d), jnp.float32),
            pltpu.SemaphoreType.DMA,
        ],
    )
    def k(h_hbm, src_hbm, dst_hbm, norm_hbm, out_hbm,
          src_v, dst_v, norm_v, rows_v, acc_sh, sem):
        c = lax.axis_index("c")
        s = lax.axis_index("s")
        wid = c * NS + s
        pltpu.sync_copy(src_hbm.at[wid], src_v)
        pltpu.sync_copy(dst_hbm.at[wid], dst_v)
        pltpu.sync_copy(norm_hbm.at[wid], norm_v)

        @pl.loop(0, K)
        def _(r):
            for q in range(d // LANES):
                rows_v[r, pl.ds(q * LANES, LANES)] = jnp.zeros((LANES,), jnp.float32)

        @pl.loop(0, rows_per_tile, step=K)
        def _(r):
            pltpu.sync_copy(rows_v, acc_sh.at[pl.ds(s * rows_per_tile + r, K), :])

        plsc.subcore_barrier()

        @pl.loop(0, nb)
        def _(j):
            pltpu.async_copy(h_hbm.at[src_v.at[j]], rows_v, sem).wait()

            @pl.loop(0, K, step=LANES)
            def _(i):
                nv = norm_v[j, pl.ds(i, LANES)]
                for t in range(LANES):
                    sc = nv[t]
                    for q in range(d // LANES):
                        sl = pl.ds(q * LANES, LANES)
                        rows_v[i + t, sl] = rows_v[i + t, sl] * sc

            pltpu.sync_copy(rows_v, acc_sh.at[dst_v.at[j]], add=True)

        plsc.subcore_barrier()
        pltpu.sync_copy(
            acc_sh.at[pl.ds(s * rows_per_tile, rows_per_tile), :],
            out_hbm.at[c, pl.ds(s * rows_per_tile, rows_per_tile), :],
        )

    return k(h, src_t, dst_t, norm_t)


def _tc_finalize_deg(deg_parts_t):
    n_pad = deg_parts_t.shape[0]

    def body(p_ref, dis_ref):
        deg = 1.0 + jnp.sum(p_ref[...], axis=1, keepdims=True)
        dis_ref[...] = lax.rsqrt(deg)

    return pl.pallas_call(
        body,
        out_shape=jax.ShapeDtypeStruct((n_pad, 1), jnp.float32),
    )(deg_parts_t)


def _tc_matmul(x, w):
    n, d_in = x.shape
    d_out = w.shape[1]
    bn = 1280

    def body(x_ref, w_ref, o_ref):
        o_ref[...] = jnp.dot(x_ref[...], w_ref[...],
                             preferred_element_type=jnp.float32)

    return pl.pallas_call(
        body,
        grid=(n // bn,),
        in_specs=[
            pl.BlockSpec((bn, d_in), lambda i: (i, 0)),
            pl.BlockSpec((d_in, d_out), lambda i: (0, 0)),
        ],
        out_specs=pl.BlockSpec((bn, d_out), lambda i: (i, 0)),
        out_shape=jax.ShapeDtypeStruct((n, d_out), jnp.float32),
    )(x, w)


def _tc_combine_mm(p0, p1, h, dis2d, b2d, w):
    n, d = h.shape
    d_out = w.shape[1]
    bn = 1280

    def body(p0_ref, p1_ref, h_ref, dis_ref, b_ref, w_ref, o_ref):
        inv_deg = dis_ref[...] * dis_ref[...]
        act = p0_ref[...] + p1_ref[...] + h_ref[...] * inv_deg + b_ref[...]
        act = jnp.maximum(act, 0.0)
        o_ref[...] = jnp.dot(act, w_ref[...], preferred_element_type=jnp.float32)

    return pl.pallas_call(
        body,
        grid=(n // bn,),
        in_specs=[
            pl.BlockSpec((bn, d), lambda i: (i, 0)),
            pl.BlockSpec((bn, d), lambda i: (i, 0)),
            pl.BlockSpec((bn, d), lambda i: (i, 0)),
            pl.BlockSpec((bn, 1), lambda i: (i, 0)),
            pl.BlockSpec((1, d), lambda i: (0, 0)),
            pl.BlockSpec((d, d_out), lambda i: (0, 0)),
        ],
        out_specs=pl.BlockSpec((bn, d_out), lambda i: (i, 0)),
        out_shape=jax.ShapeDtypeStruct((n, d_out), jnp.float32),
    )(p0, p1, h, dis2d, b2d, w)


def _tc_final(p0, p1, h, dis2d, b2d):
    n, d = h.shape
    bn = 1280

    def body(p0_ref, p1_ref, h_ref, dis_ref, b_ref, o_ref):
        inv_deg = dis_ref[...] * dis_ref[...]
        o_ref[...] = p0_ref[...] + p1_ref[...] + h_ref[...] * inv_deg + b_ref[...]

    return pl.pallas_call(
        body,
        grid=(n // bn,),
        in_specs=[
            pl.BlockSpec((bn, d), lambda i: (i, 0)),
            pl.BlockSpec((bn, d), lambda i: (i, 0)),
            pl.BlockSpec((bn, d), lambda i: (i, 0)),
            pl.BlockSpec((bn, 1), lambda i: (i, 0)),
            pl.BlockSpec((1, d), lambda i: (0, 0)),
        ],
        out_specs=pl.BlockSpec((bn, d), lambda i: (i, 0)),
        out_shape=jax.ShapeDtypeStruct((n, d), jnp.float32),
    )(p0, p1, h, dis2d, b2d)


def kernel(x, edge_index, edge_weight, W1, b1, W2, b2, W3, b3):
    n, d = x.shape
    e = edge_weight.shape[0]

    n_pad = ((n + NS * K - 1) // (NS * K)) * (NS * K)
    eq = 2 * W * NW * K
    e_pad = ((e + eq - 1) // eq) * eq
    nb = e_pad // (NW * K)

    src = edge_index[0]
    dst = edge_index[1]
    pad_e = e_pad - e
    src_t = jnp.concatenate([src, jnp.zeros((pad_e,), jnp.int32)]).reshape(NW, nb, K)
    dst_t = jnp.concatenate([dst, jnp.zeros((pad_e,), jnp.int32)]).reshape(NW, nb, K)
    ew_t = jnp.concatenate(
        [edge_weight, jnp.zeros((pad_e,), jnp.float32)]).reshape(NW, nb, K)
    x_p = jnp.pad(x, ((0, n_pad - n), (0, 0)))

    deg_parts = _sc_deg(dst_t, ew_t, n_pad)
    dis2d = _tc_finalize_deg(deg_parts.T)
    dis = dis2d.reshape(n_pad)
    norm_t = _sc_norm(src_t, dst_t, ew_t, dis, n_pad)

    b1r = b1.reshape(1, -1)
    b2r = b2.reshape(1, -1)
    b3r = b3.reshape(1, -1)

    h1 = _tc_matmul(x_p, W1)
    p = _sc_agg(h1, src_t, dst_t, norm_t, n_pad)
    h2 = _tc_combine_mm(p[0], p[1], h1, dis2d, b1r, W2)
    p = _sc_agg(h2, src_t, dst_t, norm_t, n_pad)
    h3 = _tc_combine_mm(p[0], p[1], h2, dis2d, b2r, W3)
    p = _sc_agg(h3, src_t, dst_t, norm_t, n_pad)
    out = _tc_final(p[0], p[1], h3, dis2d, b3r)
    return out[:n]


# speedup vs baseline: 1.4163x; 1.1317x over previous
import dataclasses
import functools

import jax
import jax.numpy as jnp
from jax import lax
from jax.experimental import pallas as pl
from jax.experimental.pallas import tpu as pltpu
from jax.experimental.pallas import tpu_sc as plsc

NC = 2
NS = 16
NW = NC * NS
LANES = 16
K = 128
W = 8

_MESH = plsc.VectorSubcoreMesh(core_axis_name="c", subcore_axis_name="s")

_CP = pltpu.CompilerParams()
if "needs_layout_passes" in pltpu.CompilerParams.__dataclass_fields__:
    _CP = dataclasses.replace(_CP, needs_layout_passes=False)


def _sc_deg(dst_t, ew_t, n_pad):
    nb = dst_t.shape[1]

    @functools.partial(
        pl.kernel,
        out_type=jax.ShapeDtypeStruct((NW, n_pad), jnp.float32),
        mesh=_MESH,
        compiler_params=_CP,
        scratch_types=[
            pltpu.VMEM((nb, K), jnp.int32),
            pltpu.VMEM((nb, K), jnp.float32),
            pltpu.VMEM((n_pad,), jnp.float32),
        ],
    )
    def k(dst_hbm, ew_hbm, out_hbm, dst_v, ew_v, deg_v):
        c = lax.axis_index("c")
        s = lax.axis_index("s")
        wid = c * NS + s
        pltpu.sync_copy(dst_hbm.at[wid], dst_v)
        pltpu.sync_copy(ew_hbm.at[wid], ew_v)

        @pl.loop(0, n_pad, step=LANES)
        def _(i):
            deg_v[pl.ds(i, LANES)] = jnp.zeros((LANES,), jnp.float32)

        @pl.loop(0, nb)
        def _(j):
            @pl.loop(0, K, step=LANES)
            def _(i):
                idx = dst_v[j, pl.ds(i, LANES)]
                val = ew_v[j, pl.ds(i, LANES)]
                plsc.addupdate_scatter(deg_v, [idx], val)

        pltpu.sync_copy(deg_v, out_hbm.at[wid])

    return k(dst_t, ew_t)


def _sc_norm(src_t, dst_t, ew_t, dis, n_pad):
    nb = src_t.shape[1]

    @functools.partial(
        pl.kernel,
        out_type=jax.ShapeDtypeStruct(src_t.shape, jnp.float32),
        mesh=_MESH,
        compiler_params=_CP,
        scratch_types=[
            pltpu.VMEM((nb, K), jnp.int32),
            pltpu.VMEM((nb, K), jnp.int32),
            pltpu.VMEM((nb, K), jnp.float32),
            pltpu.VMEM((n_pad,), jnp.float32),
        ],
    )
    def k(src_hbm, dst_hbm, ew_hbm, dis_hbm, out_hbm, src_v, dst_v, ew_v, dis_v):
        c = lax.axis_index("c")
        s = lax.axis_index("s")
        wid = c * NS + s
        pltpu.sync_copy(src_hbm.at[wid], src_v)
        pltpu.sync_copy(dst_hbm.at[wid], dst_v)
        pltpu.sync_copy(ew_hbm.at[wid], ew_v)
        pltpu.sync_copy(dis_hbm, dis_v)

        @pl.loop(0, nb)
        def _(j):
            @pl.loop(0, K, step=LANES)
            def _(i):
                si = src_v[j, pl.ds(i, LANES)]
                di = dst_v[j, pl.ds(i, LANES)]
                a = plsc.load_gather(dis_v, [si])
                b = plsc.load_gather(dis_v, [di])
                ew_v[j, pl.ds(i, LANES)] = a * b * ew_v[j, pl.ds(i, LANES)]

        pltpu.sync_copy(ew_v, out_hbm.at[wid])

    return k(src_t, dst_t, ew_t, dis)


def _sc_agg(h, src_t, dst_t, norm_t, n_pad):
    nb = src_t.shape[1]
    d = h.shape[1]
    rows_per_tile = n_pad // NS
    nw = nb // W

    @functools.partial(
        pl.kernel,
        out_type=jax.ShapeDtypeStruct((NC, n_pad, d), jnp.float32),
        mesh=_MESH,
        compiler_params=_CP,
        scratch_types=[
            pltpu.VMEM((W, K), jnp.int32),
            pltpu.VMEM((W, K), jnp.int32),
            pltpu.VMEM((W, K), jnp.int32),
            pltpu.VMEM((W, K), jnp.int32),
            pltpu.VMEM((W, K), jnp.float32),
            pltpu.VMEM((W, K), jnp.float32),
            pltpu.VMEM((K, d), jnp.float32),
            pltpu.VMEM((K, d), jnp.float32),
            pltpu.VMEM_SHARED((n_pad, d), jnp.float32),
            pltpu.SemaphoreType.DMA,
            pltpu.SemaphoreType.DMA,
            pltpu.SemaphoreType.DMA,
            pltpu.SemaphoreType.DMA,
            pltpu.SemaphoreType.DMA,
            pltpu.SemaphoreType.DMA,
        ],
    )
    def k(h_hbm, src_hbm, dst_hbm, norm_hbm, out_hbm,
          src_a, src_b, dst_a, dst_b, nrm_a, nrm_b, rows_a, rows_b, acc_sh,
          sga, sgb, ssa, ssb, sia, sib):
        c = lax.axis_index("c")
        s = lax.axis_index("s")
        wid = c * NS + s
        bufs = ((src_a, dst_a, nrm_a, sia), (src_b, dst_b, nrm_b, sib))
        rows = ((rows_a, sga, ssa), (rows_b, sgb, ssb))

        def idx_copies(w, parity):
            sw, dw_, nw_, si = bufs[parity]
            return (
                pltpu.make_async_copy(src_hbm.at[wid, pl.ds(w * W, W)], sw, si),
                pltpu.make_async_copy(dst_hbm.at[wid, pl.ds(w * W, W)], dw_, si),
                pltpu.make_async_copy(norm_hbm.at[wid, pl.ds(w * W, W)], nw_, si),
            )

        def fetch_idx(w, parity):
            for cp in idx_copies(w, parity):
                cp.start()

        def wait_idx(w, parity):
            for cp in idx_copies(w, parity):
                cp.wait()

        @pl.loop(0, K)
        def _(r):
            for q in range(d // LANES):
                rows_b[r, pl.ds(q * LANES, LANES)] = jnp.zeros((LANES,), jnp.float32)

        @pl.loop(0, rows_per_tile, step=K)
        def _(r):
            pltpu.sync_copy(rows_b, acc_sh.at[pl.ds(s * rows_per_tile + r, K), :])

        for cp in idx_copies(0, 0):
            cp.start()
        wait_idx(0, 0)
        fetch_idx(1, 1)
        pltpu.async_copy(h_hbm.at[src_a.at[0]], rows_a, sga)

        plsc.subcore_barrier()

        def scale(buf, nrm, b):
            @pl.loop(0, K, step=LANES)
            def _(i):
                nv = nrm[b, pl.ds(i, LANES)]
                for t in range(LANES):
                    sc = nv[t]
                    for q in range(d // LANES):
                        sl = pl.ds(q * LANES, LANES)
                        buf[i + t, sl] = buf[i + t, sl] * sc

        @pl.loop(0, nw, step=2)
        def _(w2):
            for dw in (0, 1):
                sw, dw_, nw_, _si = bufs[dw]
                for b in range(W):
                    cur, sg_c, ss_c = rows[b % 2]
                    nxt, sg_n, ss_n = rows[1 - b % 2]
                    pltpu.make_async_copy(h_hbm.at[sw.at[b]], cur, sg_c).wait()
                    if b < W - 1:
                        if b >= 1:
                            pltpu.make_async_copy(
                                nxt, acc_sh.at[dw_.at[b - 1]], ss_n).wait()
                        pltpu.async_copy(h_hbm.at[sw.at[b + 1]], nxt, sg_n)
                    scale(cur, nw_, b)
                    pltpu.async_copy(cur, acc_sh.at[dw_.at[b]], ss_c, add=True)

                pltpu.make_async_copy(rows_a, acc_sh.at[dw_.at[W - 2]], ssa).wait()
                pltpu.make_async_copy(rows_b, acc_sh.at[dw_.at[W - 1]], ssb).wait()
                if dw == 0:
                    wait_idx(w2 + 1, 1)
                    pltpu.async_copy(h_hbm.at[src_b.at[0]], rows_a, sga)

                    @pl.when(w2 + 2 < nw)
                    def _():
                        fetch_idx(w2 + 2, 0)
                else:
                    @pl.when(w2 + 2 < nw)
                    def _():
                        wait_idx(w2 + 2, 0)
                        pltpu.async_copy(h_hbm.at[src_a.at[0]], rows_a, sga)

                    @pl.when(w2 + 3 < nw)
                    def _():
                        fetch_idx(w2 + 3, 1)

        plsc.subcore_barrier()
        pltpu.sync_copy(
            acc_sh.at[pl.ds(s * rows_per_tile, rows_per_tile), :],
            out_hbm.at[c, pl.ds(s * rows_per_tile, rows_per_tile), :],
        )

    return k(h, src_t, dst_t, norm_t)


def _tc_finalize_deg(deg_parts_t):
    n_pad = deg_parts_t.shape[0]

    def body(p_ref, dis_ref):
        deg = 1.0 + jnp.sum(p_ref[...], axis=1, keepdims=True)
        dis_ref[...] = lax.rsqrt(deg)

    return pl.pallas_call(
        body,
        out_shape=jax.ShapeDtypeStruct((n_pad, 1), jnp.float32),
    )(deg_parts_t)


def _tc_matmul(x, w):
    n, d_in = x.shape
    d_out = w.shape[1]
    bn = 1280

    def body(x_ref, w_ref, o_ref):
        o_ref[...] = jnp.dot(x_ref[...], w_ref[...],
                             preferred_element_type=jnp.float32)

    return pl.pallas_call(
        body,
        grid=(n // bn,),
        in_specs=[
            pl.BlockSpec((bn, d_in), lambda i: (i, 0)),
            pl.BlockSpec((d_in, d_out), lambda i: (0, 0)),
        ],
        out_specs=pl.BlockSpec((bn, d_out), lambda i: (i, 0)),
        out_shape=jax.ShapeDtypeStruct((n, d_out), jnp.float32),
    )(x, w)


def _tc_combine_mm(p0, p1, h, dis2d, b2d, w):
    n, d = h.shape
    d_out = w.shape[1]
    bn = 1280

    def body(p0_ref, p1_ref, h_ref, dis_ref, b_ref, w_ref, o_ref):
        inv_deg = dis_ref[...] * dis_ref[...]
        act = p0_ref[...] + p1_ref[...] + h_ref[...] * inv_deg + b_ref[...]
        act = jnp.maximum(act, 0.0)
        o_ref[...] = jnp.dot(act, w_ref[...], preferred_element_type=jnp.float32)

    return pl.pallas_call(
        body,
        grid=(n // bn,),
        in_specs=[
            pl.BlockSpec((bn, d), lambda i: (i, 0)),
            pl.BlockSpec((bn, d), lambda i: (i, 0)),
            pl.BlockSpec((bn, d), lambda i: (i, 0)),
            pl.BlockSpec((bn, 1), lambda i: (i, 0)),
            pl.BlockSpec((1, d), lambda i: (0, 0)),
            pl.BlockSpec((d, d_out), lambda i: (0, 0)),
        ],
        out_specs=pl.BlockSpec((bn, d_out), lambda i: (i, 0)),
        out_shape=jax.ShapeDtypeStruct((n, d_out), jnp.float32),
    )(p0, p1, h, dis2d, b2d, w)


def _tc_final(p0, p1, h, dis2d, b2d):
    n, d = h.shape
    bn = 1280

    def body(p0_ref, p1_ref, h_ref, dis_ref, b_ref, o_ref):
        inv_deg = dis_ref[...] * dis_ref[...]
        o_ref[...] = p0_ref[...] + p1_ref[...] + h_ref[...] * inv_deg + b_ref[...]

    return pl.pallas_call(
        body,
        grid=(n // bn,),
        in_specs=[
            pl.BlockSpec((bn, d), lambda i: (i, 0)),
            pl.BlockSpec((bn, d), lambda i: (i, 0)),
            pl.BlockSpec((bn, d), lambda i: (i, 0)),
            pl.BlockSpec((bn, 1), lambda i: (i, 0)),
            pl.BlockSpec((1, d), lambda i: (0, 0)),
        ],
        out_specs=pl.BlockSpec((bn, d), lambda i: (i, 0)),
        out_shape=jax.ShapeDtypeStruct((n, d), jnp.float32),
    )(p0, p1, h, dis2d, b2d)


def kernel(x, edge_index, edge_weight, W1, b1, W2, b2, W3, b3):
    n, d = x.shape
    e = edge_weight.shape[0]

    n_pad = ((n + NS * K - 1) // (NS * K)) * (NS * K)
    eq = 2 * W * NW * K
    e_pad = ((e + eq - 1) // eq) * eq
    nb = e_pad // (NW * K)

    src = edge_index[0]
    dst = edge_index[1]
    pad_e = e_pad - e
    src_t = jnp.concatenate([src, jnp.zeros((pad_e,), jnp.int32)]).reshape(NW, nb, K)
    dst_t = jnp.concatenate([dst, jnp.zeros((pad_e,), jnp.int32)]).reshape(NW, nb, K)
    ew_t = jnp.concatenate(
        [edge_weight, jnp.zeros((pad_e,), jnp.float32)]).reshape(NW, nb, K)
    x_p = jnp.pad(x, ((0, n_pad - n), (0, 0)))

    deg_parts = _sc_deg(dst_t, ew_t, n_pad)
    dis2d = _tc_finalize_deg(deg_parts.T)
    dis = dis2d.reshape(n_pad)
    norm_t = _sc_norm(src_t, dst_t, ew_t, dis, n_pad)

    b1r = b1.reshape(1, -1)
    b2r = b2.reshape(1, -1)
    b3r = b3.reshape(1, -1)

    h1 = _tc_matmul(x_p, W1)
    p = _sc_agg(h1, src_t, dst_t, norm_t, n_pad)
    h2 = _tc_combine_mm(p[0], p[1], h1, dis2d, b1r, W2)
    p = _sc_agg(h2, src_t, dst_t, norm_t, n_pad)
    h3 = _tc_combine_mm(p[0], p[1], h2, dis2d, b2r, W3)
    p = _sc_agg(h3, src_t, dst_t, norm_t, n_pad)
    out = _tc_final(p[0], p[1], h3, dis2d, b3r)
    return out[:n]


# speedup vs baseline: 1.7872x; 1.2619x over previous
import dataclasses
import functools

import jax
import jax.numpy as jnp
from jax import lax
from jax.experimental import pallas as pl
from jax.experimental.pallas import tpu as pltpu
from jax.experimental.pallas import tpu_sc as plsc

NC = 2
NS = 16
NW = NC * NS
LANES = 16
K = 128
W = 8

_MESH = plsc.VectorSubcoreMesh(core_axis_name="c", subcore_axis_name="s")

_CP = pltpu.CompilerParams()
if "needs_layout_passes" in pltpu.CompilerParams.__dataclass_fields__:
    _CP = dataclasses.replace(_CP, needs_layout_passes=False)


def _sc_deg(dst_t, ew_t, n_pad):
    nb = dst_t.shape[1]

    @functools.partial(
        pl.kernel,
        out_type=jax.ShapeDtypeStruct((NW, n_pad), jnp.float32),
        mesh=_MESH,
        compiler_params=_CP,
        scratch_types=[
            pltpu.VMEM((nb, K), jnp.int32),
            pltpu.VMEM((nb, K), jnp.float32),
            pltpu.VMEM((n_pad,), jnp.float32),
        ],
    )
    def k(dst_hbm, ew_hbm, out_hbm, dst_v, ew_v, deg_v):
        c = lax.axis_index("c")
        s = lax.axis_index("s")
        wid = c * NS + s
        pltpu.sync_copy(dst_hbm.at[wid], dst_v)
        pltpu.sync_copy(ew_hbm.at[wid], ew_v)

        @pl.loop(0, n_pad, step=LANES)
        def _(i):
            deg_v[pl.ds(i, LANES)] = jnp.zeros((LANES,), jnp.float32)

        @pl.loop(0, nb)
        def _(j):
            @pl.loop(0, K, step=LANES)
            def _(i):
                idx = dst_v[j, pl.ds(i, LANES)]
                val = ew_v[j, pl.ds(i, LANES)]
                plsc.addupdate_scatter(deg_v, [idx], val)

        pltpu.sync_copy(deg_v, out_hbm.at[wid])

    return k(dst_t, ew_t)


def _sc_norm(src_t, dst_t, ew_t, dis, n_pad):
    nb = src_t.shape[1]

    @functools.partial(
        pl.kernel,
        out_type=jax.ShapeDtypeStruct(src_t.shape, jnp.float32),
        mesh=_MESH,
        compiler_params=_CP,
        scratch_types=[
            pltpu.VMEM((nb, K), jnp.int32),
            pltpu.VMEM((nb, K), jnp.int32),
            pltpu.VMEM((nb, K), jnp.float32),
            pltpu.VMEM((n_pad,), jnp.float32),
        ],
    )
    def k(src_hbm, dst_hbm, ew_hbm, dis_hbm, out_hbm, src_v, dst_v, ew_v, dis_v):
        c = lax.axis_index("c")
        s = lax.axis_index("s")
        wid = c * NS + s
        pltpu.sync_copy(src_hbm.at[wid], src_v)
        pltpu.sync_copy(dst_hbm.at[wid], dst_v)
        pltpu.sync_copy(ew_hbm.at[wid], ew_v)
        pltpu.sync_copy(dis_hbm, dis_v)

        @pl.loop(0, nb)
        def _(j):
            @pl.loop(0, K, step=LANES)
            def _(i):
                si = src_v[j, pl.ds(i, LANES)]
                di = dst_v[j, pl.ds(i, LANES)]
                a = plsc.load_gather(dis_v, [si])
                b = plsc.load_gather(dis_v, [di])
                ew_v[j, pl.ds(i, LANES)] = a * b * ew_v[j, pl.ds(i, LANES)]

        pltpu.sync_copy(ew_v, out_hbm.at[wid])

    return k(src_t, dst_t, ew_t, dis)


def _sc_agg(h, src_t, dst_t, norm_t, n_pad):
    nb = src_t.shape[1]
    d = h.shape[1]
    rows_per_tile = n_pad // NS

    @functools.partial(
        pl.kernel,
        out_type=jax.ShapeDtypeStruct((NC, n_pad, d), jnp.float32),
        mesh=_MESH,
        compiler_params=_CP,
        scratch_types=[
            pltpu.VMEM((nb, K), jnp.int32),
            pltpu.VMEM((nb, K), jnp.int32),
            pltpu.VMEM((nb, K), jnp.float32),
            pltpu.VMEM((K, d), jnp.float32),
            pltpu.VMEM_SHARED((n_pad, d), jnp.float32),
            pltpu.SemaphoreType.DMA,
        ],
    )
    def k(h_hbm, src_hbm, dst_hbm, norm_hbm, out_hbm,
          src_v, dst_v, norm_v, rows_v, acc_sh, sem):
        c = lax.axis_index("c")
        s = lax.axis_index("s")
        wid = c * NS + s
        pltpu.sync_copy(src_hbm.at[wid], src_v)
        pltpu.sync_copy(dst_hbm.at[wid], dst_v)
        pltpu.sync_copy(norm_hbm.at[wid], norm_v)

        @pl.loop(0, K)
        def _(r):
            for q in range(d // LANES):
                rows_v[r, pl.ds(q * LANES, LANES)] = jnp.zeros((LANES,), jnp.float32)

        @pl.loop(0, rows_per_tile, step=K)
        def _(r):
            pltpu.sync_copy(rows_v, acc_sh.at[pl.ds(s * rows_per_tile + r, K), :])

        plsc.subcore_barrier()

        @pl.loop(0, nb)
        def _(j):
            pltpu.async_copy(h_hbm.at[src_v.at[j]], rows_v, sem).wait()

            @pl.loop(0, K, step=LANES)
            def _(i):
                nv = norm_v[j, pl.ds(i, LANES)]
                for t in range(LANES):
                    sc = nv[t]
                    for q in range(d // LANES):
                        sl = pl.ds(q * LANES, LANES)
                        rows_v[i + t, sl] = rows_v[i + t, sl] * sc

            pltpu.sync_copy(rows_v, acc_sh.at[dst_v.at[j]], add=True)

        plsc.subcore_barrier()
        pltpu.sync_copy(
            acc_sh.at[pl.ds(s * rows_per_tile, rows_per_tile), :],
            out_hbm.at[c, pl.ds(s * rows_per_tile, rows_per_tile), :],
        )

    return k(h, src_t, dst_t, norm_t)


def _tc_finalize_deg(deg_parts_t):
    n_pad = deg_parts_t.shape[0]

    def body(p_ref, dis_ref):
        deg = 1.0 + jnp.sum(p_ref[...], axis=1, keepdims=True)
        dis_ref[...] = lax.rsqrt(deg)

    return pl.pallas_call(
        body,
        out_shape=jax.ShapeDtypeStruct((n_pad, 1), jnp.float32),
    )(deg_parts_t)


def _tc_matmul(x, w):
    n, d_in = x.shape
    d_out = w.shape[1]
    bn = 1280

    def body(x_ref, w_ref, o_ref):
        o_ref[...] = jnp.dot(x_ref[...], w_ref[...],
                             preferred_element_type=jnp.float32)

    return pl.pallas_call(
        body,
        grid=(n // bn,),
        in_specs=[
            pl.BlockSpec((bn, d_in), lambda i: (i, 0)),
            pl.BlockSpec((d_in, d_out), lambda i: (0, 0)),
        ],
        out_specs=pl.BlockSpec((bn, d_out), lambda i: (i, 0)),
        out_shape=jax.ShapeDtypeStruct((n, d_out), jnp.float32),
    )(x, w)


def _tc_combine_mm(p0, p1, h, dis2d, b2d, w):
    n, d = h.shape
    d_out = w.shape[1]
    bn = 1280

    def body(p0_ref, p1_ref, h_ref, dis_ref, b_ref, w_ref, o_ref):
        inv_deg = dis_ref[...] * dis_ref[...]
        act = p0_ref[...] + p1_ref[...] + h_ref[...] * inv_deg + b_ref[...]
        act = jnp.maximum(act, 0.0)
        o_ref[...] = jnp.dot(act, w_ref[...], preferred_element_type=jnp.float32)

    return pl.pallas_call(
        body,
        grid=(n // bn,),
        in_specs=[
            pl.BlockSpec((bn, d), lambda i: (i, 0)),
            pl.BlockSpec((bn, d), lambda i: (i, 0)),
            pl.BlockSpec((bn, d), lambda i: (i, 0)),
            pl.BlockSpec((bn, 1), lambda i: (i, 0)),
            pl.BlockSpec((1, d), lambda i: (0, 0)),
            pl.BlockSpec((d, d_out), lambda i: (0, 0)),
        ],
        out_specs=pl.BlockSpec((bn, d_out), lambda i: (i, 0)),
        out_shape=jax.ShapeDtypeStruct((n, d_out), jnp.float32),
    )(p0, p1, h, dis2d, b2d, w)


def _tc_final(p0, p1, h, dis2d, b2d):
    n, d = h.shape
    bn = 1280

    def body(p0_ref, p1_ref, h_ref, dis_ref, b_ref, o_ref):
        inv_deg = dis_ref[...] * dis_ref[...]
        o_ref[...] = p0_ref[...] + p1_ref[...] + h_ref[...] * inv_deg + b_ref[...]

    return pl.pallas_call(
        body,
        grid=(n // bn,),
        in_specs=[
            pl.BlockSpec((bn, d), lambda i: (i, 0)),
            pl.BlockSpec((bn, d), lambda i: (i, 0)),
            pl.BlockSpec((bn, d), lambda i: (i, 0)),
            pl.BlockSpec((bn, 1), lambda i: (i, 0)),
            pl.BlockSpec((1, d), lambda i: (0, 0)),
        ],
        out_specs=pl.BlockSpec((bn, d), lambda i: (i, 0)),
        out_shape=jax.ShapeDtypeStruct((n, d), jnp.float32),
    )(p0, p1, h, dis2d, b2d)


def kernel(x, edge_index, edge_weight, W1, b1, W2, b2, W3, b3):
    n, d = x.shape
    e = edge_weight.shape[0]

    n_pad = ((n + NS * K - 1) // (NS * K)) * (NS * K)
    e_pad = ((e + NW * K - 1) // (NW * K)) * (NW * K)
    nb = e_pad // (NW * K)

    src = edge_index[0]
    dst = edge_index[1]
    pad_e = e_pad - e
    src_t = jnp.concatenate([src, jnp.zeros((pad_e,), jnp.int32)]).reshape(NW, nb, K)
    dst_t = jnp.concatenate([dst, jnp.zeros((pad_e,), jnp.int32)]).reshape(NW, nb, K)
    ew_t = jnp.concatenate(
        [edge_weight, jnp.zeros((pad_e,), jnp.float32)]).reshape(NW, nb, K)
    x_p = jnp.pad(x, ((0, n_pad - n), (0, 0)))

    deg_parts = _sc_deg(dst_t, ew_t, n_pad)
    dis2d = _tc_finalize_deg(deg_parts.T)
    dis = dis2d.reshape(n_pad)
    norm_t = _sc_norm(src_t, dst_t, ew_t, dis, n_pad)

    b1r = b1.reshape(1, -1)
    b2r = b2.reshape(1, -1)
    b3r = b3.reshape(1, -1)

    h1 = _tc_matmul(x_p, W1)
    p = _sc_agg(h1, src_t, dst_t, norm_t, n_pad)
    h2 = _tc_combine_mm(p[0], p[1], h1, dis2d, b1r, W2)
    p = _sc_agg(h2, src_t, dst_t, norm_t, n_pad)
    h3 = _tc_combine_mm(p[0], p[1], h2, dis2d, b2r, W3)
    p = _sc_agg(h3, src_t, dst_t, norm_t, n_pad)
    out = _tc_final(p[0], p[1], h3, dis2d, b3r)
    return out[:n]
